# baseline (speedup 1.0000x reference)
import jax
import jax.numpy as jnp
from jax import lax
from jax.experimental import pallas as pl
from jax.experimental.pallas import tpu as pltpu
from jax.experimental.pallas import tpu_sc as plsc

_FIELD = 38461
_F = 26
_D = 16
_B = 16384
_NC = 2
_NS = 16
_NW = _NC * _NS
_PER_W = _B // _NW
_R = 64
_NCHUNK = _PER_W // _R
_NBUF = 2

_V = 999987
_RB = 1024
_VP8 = 131072
_TCG = _VP8 // _RB
_NSTRIPE = 8
_VP = _NSTRIPE * _VP8
_VB = _VP // _TCG


def _relayout_body(*refs):
    stripes = refs[:_NSTRIPE]
    lint_ref = refs[_NSTRIPE]
    emb_out_ref = refs[_NSTRIPE + 1]
    lin_out_ref = refs[_NSTRIPE + 2]
    eye = jnp.eye(_D, dtype=jnp.float32)
    cols = [
        jax.lax.dot_general(s[...], eye, (((0,), (0,)), ((), ())),
                            preferred_element_type=jnp.float32)
        for s in stripes
    ]
    emb_out_ref[...] = jnp.concatenate(cols, axis=1)
    lin_out_ref[...] = lint_ref[0, :]


def _fm_body(x_hbm, lin_hbm, emb_hbm, bias_hbm, out_hbm,
             xraw, ibuf, ilbuf, ebuf, lbuf, obuf, bbuf, sem_e, sem_l):
    wid = lax.axis_index("s") * _NC + lax.axis_index("c")
    base = wid * _PER_W

    pltpu.sync_copy(x_hbm.at[pl.ds(base, _PER_W), :], xraw)
    pltpu.sync_copy(bias_hbm, bbuf)

    lane = lax.iota(jnp.int32, _D)

    def tr_body(f, carry):
        off = f * _FIELD
        fvec = jnp.full((_D,), f, jnp.int32)
        for g in range(_PER_W // _D):
            rows = g * _D + lane
            vals = plsc.load_gather(xraw, [rows, fvec]) + off
            sl = pl.ds(g * _D, _D)
            ilbuf[f, sl] = vals
            ibuf[f, sl] = ((vals & (_VP8 - 1)) << 3) | (vals >> 17)
        return carry

    lax.fori_loop(0, _F, tr_body, 0)

    bval = bbuf[...]

    def fire(c, slot):
        col = pl.ds(c * _R, _R)

        def fire_body(f, carry2):
            pltpu.async_copy(
                emb_hbm.at[ibuf.at[f, col]], ebuf.at[slot, f], sem_e)
            pltpu.async_copy(
                lin_hbm.at[ilbuf.at[f, col]], lbuf.at[slot, f], sem_l)
            return carry2

        lax.fori_loop(0, _F, fire_body, 0)

    def drain():
        def drain_body(f, carry2):
            pltpu.make_async_copy(
                emb_hbm.at[pl.ds(0, _R)], ebuf.at[0, 0], sem_e).wait()
            pltpu.make_async_copy(
                lin_hbm.at[pl.ds(0, _R)], lbuf.at[0, 0], sem_l).wait()
            return carry2

        lax.fori_loop(0, _F, drain_body, 0)

    def compute(c, slot):
        def grp_body(g, carry2):
            fmvec = jnp.zeros((_D,), jnp.float32)
            for j in range(_D):
                r = g * _D + j
                s = ebuf[slot, 0, r]
                q = s * s
                for f in range(1, _F):
                    v = ebuf[slot, f, r]
                    s = s + v
                    q = q + v * v
                fm = 0.5 * jnp.sum(s * s - q)
                fmvec = jnp.where(lane == j, fm, fmvec)
            sl = pl.ds(g * _D, _D)
            lin = lbuf[slot, 0, sl]
            for f in range(1, _F):
                lin = lin + lbuf[slot, f, sl]
            z = lin + fmvec + bval
            obuf[pl.ds(c * _R + g * _D, _D)] = 1.0 / (1.0 + jnp.exp(-z))
            return carry2

        lax.fori_loop(0, _R // _D, grp_body, 0)

    fire(0, 0)

    def chunk_body(c, carry):
        nxt = c + 1

        @pl.when(nxt < _NCHUNK)
        def _():
            fire(nxt, nxt % _NBUF)

        drain()
        compute(c, c % _NBUF)
        return carry

    lax.fori_loop(0, _NCHUNK, chunk_body, 0)

    pltpu.sync_copy(obuf, out_hbm.at[pl.ds(base, _PER_W)])


@jax.jit
def kernel(x, linear_w, emb_w, bias):
    embt = emb_w.T
    lint = linear_w.T
    bias_v = jnp.broadcast_to(bias.reshape(()), (_D,))

    n_eb = _V // _RB
    n_lb = _V // _VB

    emb2d, lin1d = pl.pallas_call(
        _relayout_body,
        grid=(_TCG,),
        in_specs=[
            pl.BlockSpec(
                (_D, _RB),
                lambda g, kk=k: (0, jnp.minimum(g + kk * _TCG, n_eb)))
            for k in range(_NSTRIPE)
        ] + [
            pl.BlockSpec((1, _VB), lambda g: (0, jnp.minimum(g, n_lb))),
        ],
        out_specs=[
            pl.BlockSpec((_RB, _NSTRIPE * _D), lambda g: (g, 0)),
            pl.BlockSpec((_VB,), lambda g: (g,)),
        ],
        out_shape=[
            jax.ShapeDtypeStruct((_VP8, _NSTRIPE * _D), jnp.float32),
            jax.ShapeDtypeStruct((_VP,), jnp.float32),
        ],
    )(*([embt] * _NSTRIPE), lint)
    emb_rm = emb2d.reshape(_VP, _D)

    mesh = plsc.VectorSubcoreMesh(
        core_axis_name="c", subcore_axis_name="s",
        num_cores=_NC, num_subcores=_NS)

    fm = pl.kernel(
        _fm_body,
        out_type=jax.ShapeDtypeStruct((_B,), jnp.float32),
        mesh=mesh,
        scratch_types=[
            pltpu.VMEM((_PER_W, _F), jnp.int32),
            pltpu.VMEM((_F, _PER_W), jnp.int32),
            pltpu.VMEM((_F, _PER_W), jnp.int32),
            pltpu.VMEM((_NBUF, _F, _R, _D), jnp.float32),
            pltpu.VMEM((_NBUF, _F, _R), jnp.float32),
            pltpu.VMEM((_PER_W,), jnp.float32),
            pltpu.VMEM((_D,), jnp.float32),
            pltpu.SemaphoreType.DMA,
            pltpu.SemaphoreType.DMA,
        ],
        compiler_params=pltpu.CompilerParams(
            needs_layout_passes=False, use_tc_tiling_on_sc=False),
    )
    return fm(x.astype(jnp.int32), lin1d, emb_rm, bias_v)


# speedup vs baseline: 2.0207x; 2.0207x over previous
import jax
import jax.numpy as jnp
from jax import lax
from jax.experimental import pallas as pl
from jax.experimental.pallas import tpu as pltpu
from jax.experimental.pallas import tpu_sc as plsc

_FIELD = 38461
_F = 26
_D = 16
_B = 16384
_NC = 2
_NS = 16
_NW = _NC * _NS
_PER_W = _B // _NW
_R = 64
_NCHUNK = _PER_W // _R
_NBUF = 2

_V = 999987
_RB = 1024
_VP8 = 131072
_TCG = _VP8 // _RB
_NSTRIPE = 8
_VP = _NSTRIPE * _VP8
_VB = _VP // _TCG


def _relayout_body(*refs):
    stripes = refs[:_NSTRIPE]
    lint_ref = refs[_NSTRIPE]
    emb_out_ref = refs[_NSTRIPE + 1]
    lin_out_ref = refs[_NSTRIPE + 2]
    y = jnp.concatenate([s[...] for s in stripes], axis=0)
    emb_out_ref[...] = y.T
    lin_out_ref[...] = lint_ref[0, :]


def _fm_body(x_hbm, lin_hbm, emb_hbm, bias_hbm, out_hbm,
             xraw, ibuf, ilbuf, ebuf, lbuf, obuf, bbuf, sem_e, sem_l):
    wid = lax.axis_index("s") * _NC + lax.axis_index("c")
    base = wid * _PER_W

    pltpu.sync_copy(x_hbm.at[pl.ds(base, _PER_W), :], xraw)
    pltpu.sync_copy(bias_hbm, bbuf)

    lane = lax.iota(jnp.int32, _D)

    def tr_body(f, carry):
        off = f * _FIELD
        fvec = jnp.full((_D,), f, jnp.int32)
        for g in range(_PER_W // _D):
            rows = g * _D + lane
            vals = plsc.load_gather(xraw, [rows, fvec]) + off
            sl = pl.ds(g * _D, _D)
            ilbuf[f, sl] = vals
            ibuf[f, sl] = ((vals & (_VP8 - 1)) << 3) | (vals >> 17)
        return carry

    lax.fori_loop(0, _F, tr_body, 0)

    bval = bbuf[...]

    def fire(c, slot):
        col = pl.ds(c * _R, _R)

        def fire_body(f, carry2):
            pltpu.async_copy(
                emb_hbm.at[ibuf.at[f, col]], ebuf.at[slot, f], sem_e)
            pltpu.async_copy(
                lin_hbm.at[ilbuf.at[f, col]], lbuf.at[slot, f], sem_l)
            return carry2

        lax.fori_loop(0, _F, fire_body, 0)

    def drain():
        def drain_body(f, carry2):
            pltpu.make_async_copy(
                emb_hbm.at[pl.ds(0, _R)], ebuf.at[0, 0], sem_e).wait()
            pltpu.make_async_copy(
                lin_hbm.at[pl.ds(0, _R)], lbuf.at[0, 0], sem_l).wait()
            return carry2

        lax.fori_loop(0, _F, drain_body, 0)

    def compute(c, slot):
        def grp_body(g, carry2):
            fmvec = jnp.zeros((_D,), jnp.float32)
            for j in range(_D):
                r = g * _D + j
                s = ebuf[slot, 0, r]
                q = s * s
                for f in range(1, _F):
                    v = ebuf[slot, f, r]
                    s = s + v
                    q = q + v * v
                fm = 0.5 * jnp.sum(s * s - q)
                fmvec = jnp.where(lane == j, fm, fmvec)
            sl = pl.ds(g * _D, _D)
            lin = lbuf[slot, 0, sl]
            for f in range(1, _F):
                lin = lin + lbuf[slot, f, sl]
            z = lin + fmvec + bval
            obuf[pl.ds(c * _R + g * _D, _D)] = 1.0 / (1.0 + jnp.exp(-z))
            return carry2

        lax.fori_loop(0, _R // _D, grp_body, 0)

    fire(0, 0)

    def chunk_body(c, carry):
        nxt = c + 1

        @pl.when(nxt < _NCHUNK)
        def _():
            fire(nxt, nxt % _NBUF)

        drain()
        compute(c, c % _NBUF)
        return carry

    lax.fori_loop(0, _NCHUNK, chunk_body, 0)

    pltpu.sync_copy(obuf, out_hbm.at[pl.ds(base, _PER_W)])


@jax.jit
def kernel(x, linear_w, emb_w, bias):
    embt = emb_w.T
    lint = linear_w.T
    bias_v = jnp.broadcast_to(bias.reshape(()), (_D,))

    n_eb = _V // _RB
    n_lb = _V // _VB

    emb2d, lin1d = pl.pallas_call(
        _relayout_body,
        grid=(_TCG,),
        in_specs=[
            pl.BlockSpec(
                (_D, _RB),
                lambda g, kk=k: (0, jnp.minimum(g + kk * _TCG, n_eb)))
            for k in range(_NSTRIPE)
        ] + [
            pl.BlockSpec((1, _VB), lambda g: (0, jnp.minimum(g, n_lb))),
        ],
        out_specs=[
            pl.BlockSpec((_RB, _NSTRIPE * _D), lambda g: (g, 0)),
            pl.BlockSpec((_VB,), lambda g: (g,)),
        ],
        out_shape=[
            jax.ShapeDtypeStruct((_VP8, _NSTRIPE * _D), jnp.float32),
            jax.ShapeDtypeStruct((_VP,), jnp.float32),
        ],
    )(*([embt] * _NSTRIPE), lint)
    emb_rm = emb2d.reshape(_VP, _D)

    mesh = plsc.VectorSubcoreMesh(
        core_axis_name="c", subcore_axis_name="s",
        num_cores=_NC, num_subcores=_NS)

    fm = pl.kernel(
        _fm_body,
        out_type=jax.ShapeDtypeStruct((_B,), jnp.float32),
        mesh=mesh,
        scratch_types=[
            pltpu.VMEM((_PER_W, _F), jnp.int32),
            pltpu.VMEM((_F, _PER_W), jnp.int32),
            pltpu.VMEM((_F, _PER_W), jnp.int32),
            pltpu.VMEM((_NBUF, _F, _R, _D), jnp.float32),
            pltpu.VMEM((_NBUF, _F, _R), jnp.float32),
            pltpu.VMEM((_PER_W,), jnp.float32),
            pltpu.VMEM((_D,), jnp.float32),
            pltpu.SemaphoreType.DMA,
            pltpu.SemaphoreType.DMA,
        ],
        compiler_params=pltpu.CompilerParams(
            needs_layout_passes=False, use_tc_tiling_on_sc=False),
    )
    return fm(x.astype(jnp.int32), lin1d, emb_rm, bias_v)


# speedup vs baseline: 2.7370x; 1.3545x over previous
import jax
import jax.numpy as jnp
from jax import lax
from jax.experimental import pallas as pl
from jax.experimental.pallas import tpu as pltpu
from jax.experimental.pallas import tpu_sc as plsc

_FIELD = 38461
_F = 26
_D = 16
_B = 16384
_NC = 2
_NS = 16
_NW = _NC * _NS
_PER_W = _B // _NW
_R = 64
_NCHUNK = _PER_W // _R
_NBUF = 2

_V = 999987
_RB = 4096
_VP8 = 131072
_TCG = _VP8 // _RB
_NSTRIPE = 8
_VP = _NSTRIPE * _VP8
_VB = _VP // _TCG


def _relayout_body(*refs):
    stripes = refs[:_NSTRIPE]
    lint_ref = refs[_NSTRIPE]
    emb_out_ref = refs[_NSTRIPE + 1]
    lin_out_ref = refs[_NSTRIPE + 2]
    y = jnp.concatenate([s[...] for s in stripes], axis=0)
    emb_out_ref[...] = y.T
    lin_out_ref[...] = lint_ref[0, :]


def _fm_body(x_hbm, lin_hbm, emb_hbm, bias_hbm, out_hbm,
             xraw, ibuf, ilbuf, ebuf, lbuf, obuf, bbuf, sem_e, sem_l):
    wid = lax.axis_index("s") * _NC + lax.axis_index("c")
    base = wid * _PER_W

    pltpu.sync_copy(x_hbm.at[pl.ds(base, _PER_W), :], xraw)
    pltpu.sync_copy(bias_hbm, bbuf)

    lane = lax.iota(jnp.int32, _D)

    def tr_body(f, carry):
        off = f * _FIELD
        fvec = jnp.full((_D,), f, jnp.int32)
        for g in range(_PER_W // _D):
            rows = g * _D + lane
            vals = plsc.load_gather(xraw, [rows, fvec]) + off
            sl = pl.ds(g * _D, _D)
            ilbuf[f, sl] = vals
            ibuf[f, sl] = ((vals & (_VP8 - 1)) << 3) | (vals >> 17)
        return carry

    lax.fori_loop(0, _F, tr_body, 0)

    bval = bbuf[...]

    def fire(c, slot):
        col = pl.ds(c * _R, _R)

        def fire_body(f, carry2):
            pltpu.async_copy(
                emb_hbm.at[ibuf.at[f, col]], ebuf.at[slot, f], sem_e)
            pltpu.async_copy(
                lin_hbm.at[ilbuf.at[f, col]], lbuf.at[slot, f], sem_l)
            return carry2

        lax.fori_loop(0, _F, fire_body, 0)

    def drain():
        def drain_body(f, carry2):
            pltpu.make_async_copy(
                emb_hbm.at[pl.ds(0, _R)], ebuf.at[0, 0], sem_e).wait()
            pltpu.make_async_copy(
                lin_hbm.at[pl.ds(0, _R)], lbuf.at[0, 0], sem_l).wait()
            return carry2

        lax.fori_loop(0, _F, drain_body, 0)

    def compute(c, slot):
        def grp_body(g, carry2):
            fmvec = jnp.zeros((_D,), jnp.float32)
            for j in range(_D):
                r = g * _D + j
                s = ebuf[slot, 0, r]
                q = s * s
                for f in range(1, _F):
                    v = ebuf[slot, f, r]
                    s = s + v
                    q = q + v * v
                fm = 0.5 * jnp.sum(s * s - q)
                fmvec = jnp.where(lane == j, fm, fmvec)
            sl = pl.ds(g * _D, _D)
            lin = lbuf[slot, 0, sl]
            for f in range(1, _F):
                lin = lin + lbuf[slot, f, sl]
            z = lin + fmvec + bval
            obuf[pl.ds(c * _R + g * _D, _D)] = 1.0 / (1.0 + jnp.exp(-z))
            return carry2

        lax.fori_loop(0, _R // _D, grp_body, 0)

    fire(0, 0)

    def chunk_body(c, carry):
        nxt = c + 1

        @pl.when(nxt < _NCHUNK)
        def _():
            fire(nxt, nxt % _NBUF)

        drain()
        compute(c, c % _NBUF)
        return carry

    lax.fori_loop(0, _NCHUNK, chunk_body, 0)

    pltpu.sync_copy(obuf, out_hbm.at[pl.ds(base, _PER_W)])


@jax.jit
def kernel(x, linear_w, emb_w, bias):
    embt = emb_w.T
    lint = linear_w.T
    bias_v = jnp.broadcast_to(bias.reshape(()), (_D,))

    n_eb = _V // _RB
    n_lb = _V // _VB

    emb2d, lin1d = pl.pallas_call(
        _relayout_body,
        grid=(_TCG,),
        in_specs=[
            pl.BlockSpec(
                (_D, _RB),
                lambda g, kk=k: (0, jnp.minimum(g + kk * _TCG, n_eb)))
            for k in range(_NSTRIPE)
        ] + [
            pl.BlockSpec((1, _VB), lambda g: (0, jnp.minimum(g, n_lb))),
        ],
        out_specs=[
            pl.BlockSpec((_RB, _NSTRIPE * _D), lambda g: (g, 0)),
            pl.BlockSpec((_VB,), lambda g: (g,)),
        ],
        out_shape=[
            jax.ShapeDtypeStruct((_VP8, _NSTRIPE * _D), jnp.float32),
            jax.ShapeDtypeStruct((_VP,), jnp.float32),
        ],
    )(*([embt] * _NSTRIPE), lint)
    emb_rm = emb2d.reshape(_VP, _D)

    mesh = plsc.VectorSubcoreMesh(
        core_axis_name="c", subcore_axis_name="s",
        num_cores=_NC, num_subcores=_NS)

    fm = pl.kernel(
        _fm_body,
        out_type=jax.ShapeDtypeStruct((_B,), jnp.float32),
        mesh=mesh,
        scratch_types=[
            pltpu.VMEM((_PER_W, _F), jnp.int32),
            pltpu.VMEM((_F, _PER_W), jnp.int32),
            pltpu.VMEM((_F, _PER_W), jnp.int32),
            pltpu.VMEM((_NBUF, _F, _R, _D), jnp.float32),
            pltpu.VMEM((_NBUF, _F, _R), jnp.float32),
            pltpu.VMEM((_PER_W,), jnp.float32),
            pltpu.VMEM((_D,), jnp.float32),
            pltpu.SemaphoreType.DMA,
            pltpu.SemaphoreType.DMA,
        ],
        compiler_params=pltpu.CompilerParams(
            needs_layout_passes=False, use_tc_tiling_on_sc=False),
    )
    return fm(x.astype(jnp.int32), lin1d, emb_rm, bias_v)


# speedup vs baseline: 3.0516x; 1.1149x over previous
import jax
import jax.numpy as jnp
from jax import lax
from jax.experimental import pallas as pl
from jax.experimental.pallas import tpu as pltpu
from jax.experimental.pallas import tpu_sc as plsc

_FIELD = 38461
_F = 26
_D = 16
_B = 16384
_NC = 2
_NS = 16
_NW = _NC * _NS
_PER_W = _B // _NW
_R = 32
_NCHUNK = _PER_W // _R
_NBUF = 2

_V = 999987
_RB = 4096
_VP8 = 131072
_TCG = _VP8 // _RB
_NSTRIPE = 8
_VP = _NSTRIPE * _VP8
_VB = _VP // _TCG


def _relayout_body(*refs):
    stripes = refs[:_NSTRIPE]
    lint_ref = refs[_NSTRIPE]
    emb_out_ref = refs[_NSTRIPE + 1]
    lin_out_ref = refs[_NSTRIPE + 2]
    y = jnp.concatenate([s[...] for s in stripes], axis=0)
    emb_out_ref[...] = y.T
    lin_out_ref[...] = lint_ref[0, :]


def _fm_body(x_hbm, lin_hbm, emb_hbm, bias_hbm, out_hbm,
             xraw, ibuf, ilbuf, ebuf, lbuf, obuf, bbuf, lshared, sem_e, sem_l):
    wid = lax.axis_index("s") * _NC + lax.axis_index("c")
    base = wid * _PER_W

    pltpu.sync_copy(x_hbm.at[pl.ds(base, _PER_W), :], xraw)
    pltpu.sync_copy(bias_hbm, bbuf)

    sub = lax.axis_index("s")
    lsl = pl.ds(sub * (_VP // _NS), _VP // _NS)
    pltpu.sync_copy(lin_hbm.at[lsl], lshared.at[lsl])

    lane = lax.iota(jnp.int32, _D)

    def prep(c, slot):
        def tr_body(f, carry):
            off = f * _FIELD
            fvec = jnp.full((_D,), f, jnp.int32)
            for g in range(_R // _D):
                rows = c * _R + g * _D + lane
                vals = plsc.load_gather(xraw, [rows, fvec]) + off
                sl = pl.ds(g * _D, _D)
                ilbuf[slot, f, sl] = vals
                ibuf[slot, f, sl] = ((vals & (_VP8 - 1)) << 3) | (vals >> 17)
            return carry

        lax.fori_loop(0, _F, tr_body, 0)

    plsc.subcore_barrier()

    bval = bbuf[...]

    def fire(c, slot):
        def fire_body(f, carry2):
            pltpu.async_copy(
                emb_hbm.at[ibuf.at[slot, f]], ebuf.at[slot, f], sem_e)
            pltpu.async_copy(
                lshared.at[ilbuf.at[slot, f]], lbuf.at[slot, f], sem_l)
            return carry2

        lax.fori_loop(0, _F, fire_body, 0)

    def drain():
        def drain_body(f, carry2):
            pltpu.make_async_copy(
                emb_hbm.at[pl.ds(0, _R)], ebuf.at[0, 0], sem_e).wait()
            pltpu.make_async_copy(
                lin_hbm.at[pl.ds(0, _R)], lbuf.at[0, 0], sem_l).wait()
            return carry2

        lax.fori_loop(0, _F, drain_body, 0)

    def compute(c, slot):
        def grp_body(g, carry2):
            fmvec = jnp.zeros((_D,), jnp.float32)
            for j in range(_D):
                r = g * _D + j
                s = ebuf[slot, 0, r]
                q = s * s
                for f in range(1, _F):
                    v = ebuf[slot, f, r]
                    s = s + v
                    q = q + v * v
                fm = 0.5 * jnp.sum(s * s - q)
                fmvec = jnp.where(lane == j, fm, fmvec)
            sl = pl.ds(g * _D, _D)
            lin = lbuf[slot, 0, sl]
            for f in range(1, _F):
                lin = lin + lbuf[slot, f, sl]
            z = lin + fmvec + bval
            obuf[pl.ds(c * _R + g * _D, _D)] = 1.0 / (1.0 + jnp.exp(-z))
            return carry2

        lax.fori_loop(0, _R // _D, grp_body, 0)

    prep(0, 0)
    fire(0, 0)

    def chunk_body(c, carry):
        nxt = c + 1

        @pl.when(nxt < _NCHUNK)
        def _():
            prep(nxt, nxt % _NBUF)
            fire(nxt, nxt % _NBUF)

        drain()
        compute(c, c % _NBUF)
        return carry

    lax.fori_loop(0, _NCHUNK, chunk_body, 0)

    pltpu.sync_copy(obuf, out_hbm.at[pl.ds(base, _PER_W)])


@jax.jit
def kernel(x, linear_w, emb_w, bias):
    embt = emb_w.T
    lint = linear_w.T
    bias_v = jnp.broadcast_to(bias.reshape(()), (_D,))

    n_eb = _V // _RB
    n_lb = _V // _VB

    emb2d, lin1d = pl.pallas_call(
        _relayout_body,
        grid=(_TCG,),
        in_specs=[
            pl.BlockSpec(
                (_D, _RB),
                lambda g, kk=k: (0, jnp.minimum(g + kk * _TCG, n_eb)))
            for k in range(_NSTRIPE)
        ] + [
            pl.BlockSpec((1, _VB), lambda g: (0, jnp.minimum(g, n_lb))),
        ],
        out_specs=[
            pl.BlockSpec((_RB, _NSTRIPE * _D), lambda g: (g, 0)),
            pl.BlockSpec((_VB,), lambda g: (g,)),
        ],
        out_shape=[
            jax.ShapeDtypeStruct((_VP8, _NSTRIPE * _D), jnp.float32),
            jax.ShapeDtypeStruct((_VP,), jnp.float32),
        ],
    )(*([embt] * _NSTRIPE), lint)
    emb_rm = emb2d.reshape(_VP, _D)

    mesh = plsc.VectorSubcoreMesh(
        core_axis_name="c", subcore_axis_name="s",
        num_cores=_NC, num_subcores=_NS)

    fm = pl.kernel(
        _fm_body,
        out_type=jax.ShapeDtypeStruct((_B,), jnp.float32),
        mesh=mesh,
        scratch_types=[
            pltpu.VMEM((_PER_W, _F), jnp.int32),
            pltpu.VMEM((_NBUF, _F, _R), jnp.int32),
            pltpu.VMEM((_NBUF, _F, _R), jnp.int32),
            pltpu.VMEM((_NBUF, _F, _R, _D), jnp.float32),
            pltpu.VMEM((_NBUF, _F, _R), jnp.float32),
            pltpu.VMEM((_PER_W,), jnp.float32),
            pltpu.VMEM((_D,), jnp.float32),
            pltpu.VMEM_SHARED((_VP,), jnp.float32),
            pltpu.SemaphoreType.DMA,
            pltpu.SemaphoreType.DMA,
        ],
        compiler_params=pltpu.CompilerParams(
            needs_layout_passes=False, use_tc_tiling_on_sc=False),
    )
    return fm(x.astype(jnp.int32), lin1d, emb_rm, bias_v)


# speedup vs baseline: 3.2656x; 1.0701x over previous
import jax
import jax.numpy as jnp
from jax import lax
from jax.experimental import pallas as pl
from jax.experimental.pallas import tpu as pltpu
from jax.experimental.pallas import tpu_sc as plsc

_FIELD = 38461
_F = 26
_D = 16
_B = 16384
_NC = 2
_NS = 16
_NW = _NC * _NS
_PER_W = _B // _NW
_R = 32
_NCHUNK = _PER_W // _R
_NBUF = 2

_V = 999987
_RB = 4096
_VP8 = 131072
_TCG = _VP8 // _RB
_NSTRIPE = 8
_VP = _NSTRIPE * _VP8
_VB = _VP // _TCG


_NXS = 4
_XBR = 128
_XSTRIDE = _B // _NXS


def _relayout_body(*refs):
    stripes = refs[:_NSTRIPE]
    lint_ref = refs[_NSTRIPE]
    xrefs = refs[_NSTRIPE + 1:_NSTRIPE + 1 + _NXS]
    emb_out_ref = refs[_NSTRIPE + 1 + _NXS]
    lin_out_ref = refs[_NSTRIPE + 2 + _NXS]
    x_out_ref = refs[_NSTRIPE + 3 + _NXS]
    y = jnp.concatenate([s[...] for s in stripes], axis=0)
    emb_out_ref[...] = y.T
    lin_out_ref[...] = lint_ref[0, :]
    z6 = jnp.zeros((_XBR, 32 - _F), jnp.int32)
    x_out_ref[...] = jnp.concatenate(
        [b for xr in xrefs for b in (xr[...], z6)], axis=1)


def _fm_body(x_hbm, lin_hbm, emb_hbm, bias_hbm, out_hbm,
             xraw, ibuf, ilbuf, ebuf, lbuf, obuf, bbuf, lshared, sem_e, sem_l):
    wid = lax.axis_index("s") * _NC + lax.axis_index("c")
    base = wid * _PER_W

    pltpu.sync_copy(
        x_hbm.at[pl.ds((wid % 8) * _PER_W, _PER_W),
                 pl.ds((wid // 8) * 32, 32)], xraw)
    pltpu.sync_copy(bias_hbm, bbuf)

    sub = lax.axis_index("s")
    lsl = pl.ds(sub * (_VP // _NS), _VP // _NS)
    pltpu.sync_copy(lin_hbm.at[lsl], lshared.at[lsl])

    lane = lax.iota(jnp.int32, _D)

    def prep(c, slot):
        def tr_body(f, carry):
            off = f * _FIELD
            fvec = jnp.full((_D,), f, jnp.int32)
            for g in range(_R // _D):
                rows = c * _R + g * _D + lane
                vals = plsc.load_gather(xraw, [rows, fvec]) + off
                sl = pl.ds(g * _D, _D)
                ilbuf[slot, f, sl] = vals
                ibuf[slot, f, sl] = ((vals & (_VP8 - 1)) << 3) | (vals >> 17)
            return carry

        lax.fori_loop(0, _F, tr_body, 0)

    plsc.subcore_barrier()

    bval = bbuf[...]

    def fire(c, slot):
        def fire_body(f, carry2):
            pltpu.async_copy(
                emb_hbm.at[ibuf.at[slot, f]], ebuf.at[slot, f], sem_e)
            pltpu.async_copy(
                lshared.at[ilbuf.at[slot, f]], lbuf.at[slot, f], sem_l)
            return carry2

        lax.fori_loop(0, _F, fire_body, 0)

    def drain():
        def drain_body(f, carry2):
            pltpu.make_async_copy(
                emb_hbm.at[pl.ds(0, _R)], ebuf.at[0, 0], sem_e).wait()
            pltpu.make_async_copy(
                lin_hbm.at[pl.ds(0, _R)], lbuf.at[0, 0], sem_l).wait()
            return carry2

        lax.fori_loop(0, _F, drain_body, 0)

    def compute(c, slot):
        def grp_body(g, carry2):
            fmvec = jnp.zeros((_D,), jnp.float32)
            for j in range(_D):
                r = g * _D + j
                s = ebuf[slot, 0, r]
                q = s * s
                for f in range(1, _F):
                    v = ebuf[slot, f, r]
                    s = s + v
                    q = q + v * v
                fm = 0.5 * jnp.sum(s * s - q)
                fmvec = jnp.where(lane == j, fm, fmvec)
            sl = pl.ds(g * _D, _D)
            lin = lbuf[slot, 0, sl]
            for f in range(1, _F):
                lin = lin + lbuf[slot, f, sl]
            z = lin + fmvec + bval
            obuf[pl.ds(c * _R + g * _D, _D)] = 1.0 / (1.0 + jnp.exp(-z))
            return carry2

        lax.fori_loop(0, _R // _D, grp_body, 0)

    prep(0, 0)
    fire(0, 0)

    def chunk_body(c, carry):
        nxt = c + 1

        @pl.when(nxt < _NCHUNK)
        def _():
            prep(nxt, nxt % _NBUF)
            fire(nxt, nxt % _NBUF)

        drain()
        compute(c, c % _NBUF)
        return carry

    lax.fori_loop(0, _NCHUNK, chunk_body, 0)

    pltpu.sync_copy(obuf, out_hbm.at[pl.ds(base, _PER_W)])


@jax.jit
def kernel(x, linear_w, emb_w, bias):
    embt = emb_w.T
    lint = linear_w.T
    bias_v = jnp.broadcast_to(bias.reshape(()), (_D,))

    n_eb = _V // _RB
    n_lb = _V // _VB

    x_i32 = x.astype(jnp.int32)
    emb2d, lin1d, x2 = pl.pallas_call(
        _relayout_body,
        grid=(_TCG,),
        in_specs=[
            pl.BlockSpec(
                (_D, _RB),
                lambda g, kk=k: (0, jnp.minimum(g + kk * _TCG, n_eb)))
            for k in range(_NSTRIPE)
        ] + [
            pl.BlockSpec((1, _VB), lambda g: (0, jnp.minimum(g, n_lb))),
        ] + [
            pl.BlockSpec((_XBR, _F), lambda g, kk=k: (g + kk * _TCG, 0))
            for k in range(_NXS)
        ],
        out_specs=[
            pl.BlockSpec((_RB, _NSTRIPE * _D), lambda g: (g, 0)),
            pl.BlockSpec((_VB,), lambda g: (g,)),
            pl.BlockSpec((_XBR, _NXS * 32), lambda g: (g, 0)),
        ],
        out_shape=[
            jax.ShapeDtypeStruct((_VP8, _NSTRIPE * _D), jnp.float32),
            jax.ShapeDtypeStruct((_VP,), jnp.float32),
            jax.ShapeDtypeStruct((_XSTRIDE, _NXS * 32), jnp.int32),
        ],
    )(*([embt] * _NSTRIPE), lint, *([x_i32] * _NXS))
    emb_rm = emb2d.reshape(_VP, _D)

    mesh = plsc.VectorSubcoreMesh(
        core_axis_name="c", subcore_axis_name="s",
        num_cores=_NC, num_subcores=_NS)

    fm = pl.kernel(
        _fm_body,
        out_type=jax.ShapeDtypeStruct((_B,), jnp.float32),
        mesh=mesh,
        scratch_types=[
            pltpu.VMEM((_PER_W, 32), jnp.int32),
            pltpu.VMEM((_NBUF, _F, _R), jnp.int32),
            pltpu.VMEM((_NBUF, _F, _R), jnp.int32),
            pltpu.VMEM((_NBUF, _F, _R, _D), jnp.float32),
            pltpu.VMEM((_NBUF, _F, _R), jnp.float32),
            pltpu.VMEM((_PER_W,), jnp.float32),
            pltpu.VMEM((_D,), jnp.float32),
            pltpu.VMEM_SHARED((_VP,), jnp.float32),
            pltpu.SemaphoreType.DMA,
            pltpu.SemaphoreType.DMA,
        ],
        compiler_params=pltpu.CompilerParams(
            needs_layout_passes=False, use_tc_tiling_on_sc=False),
    )
    return fm(x2, lin1d, emb_rm, bias_v)
